# Initial kernel scaffold; baseline (speedup 1.0000x reference)
import jax
import jax.numpy as jnp
from jax.experimental import pallas as pl


def kernel(x_nchw, w1, b1, w2, b2):
    raise NotImplementedError("write your pallas kernel here")


# speedup vs baseline: 1.1118x; 1.1118x over previous
import functools

import jax
import jax.numpy as jnp
from jax.experimental import pallas as pl
from jax.experimental.pallas import tpu as pltpu

_LANE = 128


def _se_kernel(x_ref, w1_ref, b1_ref, w2_ref, b2_ref, out_ref,
               stash_ref, acc_ref, gate_ref, *, n_tiles, thw, inv_hw):
    s = pl.program_id(1)

    @pl.when(s == 0)
    def _():
        acc_ref[...] = jnp.zeros_like(acc_ref)

    @pl.when(s < n_tiles)
    def _():
        x = x_ref[0]
        stash_ref[:, pl.ds(s * thw, thw)] = x
        part = x[:, 0:_LANE]
        for j in range(1, thw // _LANE):
            part = part + x[:, j * _LANE:(j + 1) * _LANE]
        acc_ref[...] += part

    @pl.when(s == n_tiles - 1)
    def _():
        p = jnp.sum(acc_ref[...], axis=-1, keepdims=True) * inv_hw
        h = jnp.dot(w1_ref[...], p, preferred_element_type=jnp.float32)
        h = jnp.maximum(h + b1_ref[...], 0.0)
        g = jnp.dot(w2_ref[...], h, preferred_element_type=jnp.float32)
        g = jax.nn.sigmoid(g + b2_ref[...])
        gate_ref[...] = jnp.broadcast_to(g, gate_ref.shape)

    @pl.when(s >= n_tiles - 1)
    def _():
        o = s - (n_tiles - 1)
        xt = stash_ref[:, pl.ds(o * thw, thw)]
        out_ref[0] = xt * gate_ref[:, 0:1]


def kernel(x_nchw, w1, b1, w2, b2):
    B, C, H, W = x_nchw.shape
    HW = H * W
    Cr = w1.shape[0]

    x_flat = x_nchw.reshape(B, C, HW)
    b1c = b1.reshape(Cr, 1)
    b2c = b2.reshape(C, 1)

    thw = 2048
    while HW % thw != 0:
        thw //= 2
    n_tiles = HW // thw
    steps = 2 * n_tiles - 1

    out_flat = pl.pallas_call(
        functools.partial(_se_kernel, n_tiles=n_tiles, thw=thw,
                          inv_hw=1.0 / HW),
        out_shape=jax.ShapeDtypeStruct((B, C, HW), x_nchw.dtype),
        grid_spec=pltpu.PrefetchScalarGridSpec(
            num_scalar_prefetch=0,
            grid=(B, steps),
            in_specs=[
                pl.BlockSpec((1, C, thw),
                             lambda b, s: (b, 0, jnp.minimum(s, n_tiles - 1))),
                pl.BlockSpec((Cr, C), lambda b, s: (0, 0)),
                pl.BlockSpec((Cr, 1), lambda b, s: (0, 0)),
                pl.BlockSpec((C, Cr), lambda b, s: (0, 0)),
                pl.BlockSpec((C, 1), lambda b, s: (0, 0)),
            ],
            out_specs=pl.BlockSpec(
                (1, C, thw),
                lambda b, s: (b, 0, jnp.maximum(s - (n_tiles - 1), 0))),
            scratch_shapes=[
                pltpu.VMEM((C, HW), jnp.float32),
                pltpu.VMEM((C, _LANE), jnp.float32),
                pltpu.VMEM((C, _LANE), jnp.float32),
            ],
        ),
        compiler_params=pltpu.CompilerParams(
            dimension_semantics=("parallel", "arbitrary"),
            vmem_limit_bytes=64 * 1024 * 1024),
        cost_estimate=pl.CostEstimate(
            flops=3 * B * C * HW + 4 * B * C * Cr,
            transcendentals=B * C,
            bytes_accessed=2 * B * C * HW * 4 + 2 * C * Cr * 4),
    )(x_flat, w1, b1c, w2, b2c)

    return out_flat.reshape(B, C, H, W)
